# baseline (speedup 1.0000x reference)
import functools

import jax
import jax.numpy as jnp
from jax import lax
from jax.experimental import pallas as pl
from jax.experimental.pallas import tpu as pltpu
from jax.experimental.pallas import tpu_sc as plsc

NUM_CORES = 2
NUM_SUBCORES = 16
NUM_WORKERS = NUM_CORES * NUM_SUBCORES
LANES = 16

VOCAB = 1000000
BATCH = 1024
SEQ = 200
D = 64
DPAD = 128
SPLITS = ((0, 104), (104, 96))
B_PER_W = BATCH // NUM_WORKERS
NBUF = 2

TBLK = 512


def _transpose_kernel(src_ref, dst_ref):
    t = jnp.transpose(src_ref[...])
    dst_ref[...] = jnp.concatenate([t, t], axis=1)


def _embed_kernel(ids_hbm, table_hbm, pos_hbm, out_hbm,
                  pos_v, idx_v, tok_v, stg_v, sem_g0, sem_g1, sem_o0, sem_o1):
    sems_g = (sem_g0, sem_g1)
    sems_o = (sem_o0, sem_o1)
    wid = lax.axis_index("s") * NUM_CORES + lax.axis_index("c")
    base = wid * B_PER_W

    pltpu.sync_copy(pos_hbm.at[pl.ds(0, SEQ)], pos_v)

    def stage(g):
        slot = g % NBUF
        pltpu.sync_copy(ids_hbm.at[base + g], idx_v.at[slot])
        handles = []
        for off, n in SPLITS:
            handles.append(pltpu.async_copy(
                table_hbm.at[idx_v.at[slot, pl.ds(off, n)]],
                tok_v.at[slot, pl.ds(off, n)],
                sems_g[slot]))
        return handles

    gather_h = {0: stage(0)}
    out_h = {}
    for g in range(B_PER_W):
        slot = g % NBUF
        if g + 1 < B_PER_W:
            if g + 1 - NBUF >= 0:
                out_h.pop(g + 1 - NBUF).wait()
            gather_h[g + 1] = stage(g + 1)
        for h in gather_h.pop(g):
            h.wait()

        tok = tok_v.at[slot]
        stg = stg_v.at[slot]

        def add_row(i, carry):
            for j in range(D // LANES):
                sl = pl.ds(j * LANES, LANES)
                stg[i, sl] = tok[i, sl] + pos_v[i, sl]
            return carry

        lax.fori_loop(0, SEQ, add_row, 0)

        out_h[g] = pltpu.async_copy(stg, out_hbm.at[base + g], sems_o[slot])
    for h in out_h.values():
        h.wait()


@jax.jit
def kernel(token_ids, token_table, pos_table):
    table_cm = token_table.T
    table_rm = pl.pallas_call(
        _transpose_kernel,
        grid=((VOCAB + TBLK - 1) // TBLK,),
        in_specs=[pl.BlockSpec((D, TBLK), lambda j: (0, j))],
        out_specs=pl.BlockSpec((TBLK, 2 * D), lambda j: (j, 0)),
        out_shape=jax.ShapeDtypeStruct((VOCAB, 2 * D), jnp.float32),
    )(table_cm)
    table_flat = table_rm.reshape(2 * VOCAB, D)

    mesh = plsc.VectorSubcoreMesh(core_axis_name="c", subcore_axis_name="s")
    run = functools.partial(
        pl.kernel,
        out_type=jax.ShapeDtypeStruct((BATCH, SEQ, DPAD), jnp.float32),
        mesh=mesh,
        scratch_types=[
            pltpu.VMEM((SEQ, D), jnp.float32),
            pltpu.VMEM((NBUF, SEQ), jnp.int32),
            pltpu.VMEM((NBUF, SEQ, D), jnp.float32),
            pltpu.VMEM((NBUF, SEQ, DPAD), jnp.float32),
            pltpu.SemaphoreType.DMA,
            pltpu.SemaphoreType.DMA,
            pltpu.SemaphoreType.DMA,
            pltpu.SemaphoreType.DMA,
        ],
        compiler_params=pltpu.CompilerParams(use_tc_tiling_on_sc=False),
    )(_embed_kernel)
    out = run(token_ids.astype(jnp.int32) * 2, table_flat, pos_table)
    return out[:, :, :D]


# speedup vs baseline: 2.2153x; 2.2153x over previous
import functools

import jax
import jax.numpy as jnp
from jax import lax
from jax.experimental import pallas as pl
from jax.experimental.pallas import tpu as pltpu
from jax.experimental.pallas import tpu_sc as plsc

NUM_CORES = 2
NUM_SUBCORES = 16
NUM_WORKERS = NUM_CORES * NUM_SUBCORES
LANES = 16

BATCH = 1024
SEQ = 200
D = 64
DPAD = 128
SPLITS = ((0, 104), (104, 96))
B_PER_W = BATCH // NUM_WORKERS
CHUNK = 2
NBUF = 2
NSTEPS = B_PER_W // CHUNK


def _embed_kernel(ids_hbm, table_hbm, pos_hbm, out_hbm,
                  pos_v, idx_v, tok_v, sem_g0, sem_g1, sem_o0, sem_o1):
    sems_g = (sem_g0, sem_g1)
    sems_o = (sem_o0, sem_o1)
    wid = lax.axis_index("s") * NUM_CORES + lax.axis_index("c")
    base = wid * B_PER_W

    pltpu.sync_copy(ids_hbm.at[pl.ds(base, B_PER_W)], idx_v)
    pltpu.sync_copy(pos_hbm.at[pl.ds(0, SEQ)], pos_v)

    def stage(g):
        slot = g % NBUF
        handles = []
        for c in range(CHUNK):
            b_local = g * CHUNK + c
            for off, n in SPLITS:
                handles.append(pltpu.async_copy(
                    table_hbm.at[idx_v.at[b_local, pl.ds(off, n)]],
                    tok_v.at[slot, c, pl.ds(off, n)],
                    sems_g[slot]))
        return handles

    gather_h = {0: stage(0)}
    out_h = {}
    for g in range(NSTEPS):
        slot = g % NBUF
        if g + 1 < NSTEPS:
            if g + 1 - NBUF >= 0:
                out_h.pop(g + 1 - NBUF).wait()
            gather_h[g + 1] = stage(g + 1)
        for h in gather_h.pop(g):
            h.wait()

        tok = tok_v.at[slot]

        def add_row(i, carry):
            for j in range(D // LANES):
                sl = pl.ds(j * LANES, LANES)
                p = pos_v[i, sl]
                for c in range(CHUNK):
                    plsc.addupdate(tok.at[c, i, sl], p)
            return carry

        lax.fori_loop(0, SEQ, add_row, 0)

        b0 = base + g * CHUNK
        out_h[g] = pltpu.async_copy(tok, out_hbm.at[pl.ds(b0, CHUNK)],
                                    sems_o[slot])
    for h in out_h.values():
        h.wait()


@jax.jit
def kernel(token_ids, token_table, pos_table):
    table128 = jnp.pad(token_table, ((0, 0), (0, DPAD - D)))
    mesh = plsc.VectorSubcoreMesh(core_axis_name="c", subcore_axis_name="s")
    run = functools.partial(
        pl.kernel,
        out_type=jax.ShapeDtypeStruct((BATCH, SEQ, DPAD), jnp.float32),
        mesh=mesh,
        scratch_types=[
            pltpu.VMEM((SEQ, D), jnp.float32),
            pltpu.VMEM((B_PER_W, SEQ), jnp.int32),
            pltpu.VMEM((NBUF, CHUNK, SEQ, DPAD), jnp.float32),
            pltpu.SemaphoreType.DMA,
            pltpu.SemaphoreType.DMA,
            pltpu.SemaphoreType.DMA,
            pltpu.SemaphoreType.DMA,
        ],
        compiler_params=pltpu.CompilerParams(use_tc_tiling_on_sc=False),
    )(_embed_kernel)
    out = run(token_ids.astype(jnp.int32), table128, pos_table)
    return out[:, :, :D]


# speedup vs baseline: 2.3724x; 1.0709x over previous
import functools

import jax
import jax.numpy as jnp
from jax import lax
from jax.experimental import pallas as pl
from jax.experimental.pallas import tpu as pltpu
from jax.experimental.pallas import tpu_sc as plsc

NUM_CORES = 2
NUM_SUBCORES = 16
NUM_WORKERS = NUM_CORES * NUM_SUBCORES
LANES = 16

BATCH = 1024
SEQ = 200
D = 64
DPAD = 128
SPLITS = ((0, 104), (104, 96))
B_PER_W = BATCH // NUM_WORKERS
CHUNK = 2
NBUF = 2
NSTEPS = B_PER_W // CHUNK


def _embed_kernel(ids_hbm, table_hbm, pos_hbm, out_hbm,
                  pos_v, idx_v, tok_v, sem_g0, sem_g1, sem_o0, sem_o1):
    sems_g = (sem_g0, sem_g1)
    sems_o = (sem_o0, sem_o1)
    wid = lax.axis_index("s") * NUM_CORES + lax.axis_index("c")
    base = wid * B_PER_W

    pltpu.sync_copy(ids_hbm.at[pl.ds(base, B_PER_W)], idx_v)
    pltpu.sync_copy(pos_hbm.at[pl.ds(0, SEQ)], pos_v)

    def stage(g):
        slot = g % NBUF
        handles = []
        for c in range(CHUNK):
            b_local = g * CHUNK + c
            for off, n in SPLITS:
                handles.append(pltpu.async_copy(
                    table_hbm.at[idx_v.at[b_local, pl.ds(off, n)]],
                    tok_v.at[slot, c, pl.ds(off, n)],
                    sems_g[slot]))
        return handles

    gather_h = {0: stage(0)}
    out_h = {}
    for g in range(NSTEPS):
        slot = g % NBUF
        if g + 1 < NSTEPS:
            if g + 1 - NBUF >= 0:
                out_h.pop(g + 1 - NBUF).wait()
            gather_h[g + 1] = stage(g + 1)
        for h in gather_h.pop(g):
            h.wait()

        tok = tok_v.at[slot]

        def add_row(i, carry):
            for j in range(D // LANES):
                sl = pl.ds(j * LANES, LANES)
                p = pos_v[i, sl]
                for c in range(CHUNK):
                    plsc.addupdate(tok.at[c, i, sl], p)
            return carry

        lax.fori_loop(0, SEQ, add_row, 0)

        b0 = base + g * CHUNK
        out_h[g] = pltpu.async_copy(
            tok, out_hbm.at[pl.ds(b0, CHUNK), slice(None), pl.ds(0, D)],
            sems_o[slot])
    for h in out_h.values():
        h.wait()


@jax.jit
def kernel(token_ids, token_table, pos_table):
    table128 = jnp.pad(token_table, ((0, 0), (0, DPAD - D)))
    mesh = plsc.VectorSubcoreMesh(core_axis_name="c", subcore_axis_name="s")
    run = functools.partial(
        pl.kernel,
        out_type=jax.ShapeDtypeStruct((BATCH, SEQ, DPAD), jnp.float32),
        mesh=mesh,
        scratch_types=[
            pltpu.VMEM((SEQ, D), jnp.float32),
            pltpu.VMEM((B_PER_W, SEQ), jnp.int32),
            pltpu.VMEM((NBUF, CHUNK, SEQ, D), jnp.float32),
            pltpu.SemaphoreType.DMA,
            pltpu.SemaphoreType.DMA,
            pltpu.SemaphoreType.DMA,
            pltpu.SemaphoreType.DMA,
        ],
        compiler_params=pltpu.CompilerParams(use_tc_tiling_on_sc=False),
    )(_embed_kernel)
    out = run(token_ids.astype(jnp.int32) * 2, table128.reshape(2 * 1000000, D),
              pos_table)
    return out[:, :, :D]


# speedup vs baseline: 2.3732x; 1.0003x over previous
import functools

import jax
import jax.numpy as jnp
from jax import lax
from jax.experimental import pallas as pl
from jax.experimental.pallas import tpu as pltpu
from jax.experimental.pallas import tpu_sc as plsc

NUM_CORES = 2
NUM_SUBCORES = 16
NUM_WORKERS = NUM_CORES * NUM_SUBCORES
LANES = 16

VOCAB = 1000000
BATCH = 1024
SEQ = 200
D = 64
DPAD = 128
SPLITS = ((0, 104), (104, 96))
B_PER_W = BATCH // NUM_WORKERS
CHUNK = 2
NBUF = 2
NSTEPS = B_PER_W // CHUNK


def _embed_kernel(ids_hbm, table_hbm, pos_hbm, out_hbm,
                  pos_v, idx_v, tok_v, sem_g0, sem_g1, sem_o0, sem_o1):
    sems_g = (sem_g0, sem_g1)
    sems_o = (sem_o0, sem_o1)
    wid = lax.axis_index("s") * NUM_CORES + lax.axis_index("c")
    base = wid * B_PER_W

    pltpu.sync_copy(ids_hbm.at[pl.ds(base, B_PER_W)], idx_v)
    pltpu.sync_copy(pos_hbm.at[pl.ds(0, SEQ)], pos_v)

    def stage(g):
        slot = g % NBUF
        handles = []
        for c in range(CHUNK):
            b_local = g * CHUNK + c
            for off, n in SPLITS:
                handles.append(pltpu.async_copy(
                    table_hbm.at[idx_v.at[b_local, pl.ds(off, n)]],
                    tok_v.at[slot, c, pl.ds(off, n)],
                    sems_g[slot]))
        return handles

    gather_h = {0: stage(0)}
    out_h = {}
    for g in range(NSTEPS):
        slot = g % NBUF
        if g + 1 < NSTEPS:
            if g + 1 - NBUF >= 0:
                out_h.pop(g + 1 - NBUF).wait()
            gather_h[g + 1] = stage(g + 1)
        for h in gather_h.pop(g):
            h.wait()

        tok = tok_v.at[slot]

        def add_row(i, carry):
            for j in range(D // LANES):
                sl = pl.ds(j * LANES, LANES)
                p = pos_v[i, sl]
                for c in range(CHUNK):
                    plsc.addupdate(tok.at[c, i, sl], p)
            return carry

        lax.fori_loop(0, SEQ, add_row, 0)

        b0 = base + g * CHUNK
        out_h[g] = pltpu.async_copy(
            tok, out_hbm.at[pl.ds(b0, CHUNK), slice(None), pl.ds(0, D)],
            sems_o[slot])
    for h in out_h.values():
        h.wait()


@jax.jit
def kernel(token_ids, token_table, pos_table):
    table128 = jnp.pad(token_table, ((0, 0), (0, DPAD - D)))
    mesh = plsc.VectorSubcoreMesh(core_axis_name="c", subcore_axis_name="s")
    run = functools.partial(
        pl.kernel,
        out_type=jax.ShapeDtypeStruct((BATCH, SEQ, DPAD), jnp.float32),
        mesh=mesh,
        scratch_types=[
            pltpu.VMEM((SEQ, D), jnp.float32),
            pltpu.VMEM((B_PER_W, SEQ), jnp.int32),
            pltpu.VMEM((NBUF, CHUNK, SEQ, D), jnp.float32),
            pltpu.SemaphoreType.DMA,
            pltpu.SemaphoreType.DMA,
            pltpu.SemaphoreType.DMA,
            pltpu.SemaphoreType.DMA,
        ],
        compiler_params=pltpu.CompilerParams(use_tc_tiling_on_sc=False),
    )(_embed_kernel)
    out = run(token_ids.astype(jnp.int32) * 2, table128.reshape(2 * VOCAB, D),
              pos_table)
    return out[:, :, :D]
